# Initial kernel scaffold; baseline (speedup 1.0000x reference)
import jax
import jax.numpy as jnp
from jax import lax
from jax.experimental import pallas as pl
from jax.experimental.pallas import tpu as pltpu
from jax.experimental.pallas import tpu_sc as plsc

N_FEATURES = 64
BATCH = 4096
SEQ_LEN = 200

NC = 2
NS = 16
NW = NC * NS

N_FLAT = BATCH * SEQ_LEN
NPW = N_FLAT // NW
C = 128
G = NPW // C
POS_EXT = 3200
POS_PERIOD = POS_EXT // C


def _body(x_ref, tab_ref, pos_ref, out_ref, pos_ext, idx_v, buf, sem):
    c = lax.axis_index("c")
    s = lax.axis_index("s")
    wid = s * NC + c
    pltpu.sync_copy(pos_ref, pos_ext.at[pl.ds(s * SEQ_LEN, SEQ_LEN)])
    plsc.subcore_barrier()
    base0 = wid * NPW

    @pl.loop(0, G)
    def _(g):
        base = base0 + g * C
        off = lax.rem(g, POS_PERIOD) * C
        pltpu.sync_copy(pos_ext.at[pl.ds(off, C)], buf)
        pltpu.sync_copy(x_ref.at[pl.ds(base, C)], idx_v)
        pltpu.async_copy(tab_ref.at[idx_v], buf, sem, add=True).wait()
        pltpu.sync_copy(buf, out_ref.at[pl.ds(base, C)])


@jax.jit
def _embed(x_flat, table, pos2d):
    run = pl.kernel(
        _body,
        out_type=jax.ShapeDtypeStruct((N_FLAT, N_FEATURES), jnp.float32),
        mesh=plsc.VectorSubcoreMesh(core_axis_name="c", subcore_axis_name="s"),
        scratch_types=[
            pltpu.VMEM_SHARED((POS_EXT, N_FEATURES), jnp.float32),
            pltpu.VMEM((C,), jnp.int32),
            pltpu.VMEM((C, N_FEATURES), jnp.float32),
            pltpu.SemaphoreType.DMA,
        ],
    )
    return run(x_flat, table, pos2d)


def kernel(x, text_embedding_weight, pos_embedding):
    bs, seq_len = x.shape
    x_flat = x.reshape(-1).astype(jnp.int32)
    pos2d = pos_embedding.reshape(-1, pos_embedding.shape[-1])[:seq_len]
    out = _embed(x_flat, text_embedding_weight, pos2d)
    return out.reshape(bs, seq_len, N_FEATURES)


# speedup vs baseline: 2.9657x; 2.9657x over previous
import jax
import jax.numpy as jnp
from jax import lax
from jax.experimental import pallas as pl
from jax.experimental.pallas import tpu as pltpu
from jax.experimental.pallas import tpu_sc as plsc

N_FEATURES = 64
BATCH = 4096
SEQ_LEN = 200

NC = 2
NS = 16
NW = NC * NS

N_FLAT = BATCH * SEQ_LEN
NPW = N_FLAT // NW
C = 128
G = NPW // C
POS_EXT = 3200
POS_PERIOD = POS_EXT // C


def _body(x_ref, tab_ref, pos_ref, out_ref, pos_ext, idx_v, buf, sem):
    c = lax.axis_index("c")
    s = lax.axis_index("s")
    wid = s * NC + c
    pltpu.sync_copy(pos_ref, pos_ext.at[pl.ds(s * SEQ_LEN, SEQ_LEN)])
    plsc.subcore_barrier()
    base0 = wid * NPW

    @pl.loop(0, G)
    def _(g):
        base = base0 + g * C
        off = lax.rem(g, POS_PERIOD) * C
        pltpu.sync_copy(pos_ext.at[pl.ds(off, C)], buf)
        pltpu.sync_copy(x_ref.at[pl.ds(base, C)], idx_v)
        pltpu.async_copy(tab_ref.at[idx_v], buf, sem, add=True).wait()
        pltpu.sync_copy(buf, out_ref.at[pl.ds(base, C)])


@jax.jit
def _embed(x_flat, table, pos2d):
    run = pl.kernel(
        _body,
        out_type=jax.ShapeDtypeStruct((N_FLAT, N_FEATURES), jnp.float32),
        mesh=plsc.VectorSubcoreMesh(core_axis_name="c", subcore_axis_name="s"),
        scratch_types=[
            pltpu.VMEM_SHARED((POS_EXT, N_FEATURES), jnp.float32),
            pltpu.VMEM((C,), jnp.int32),
            pltpu.VMEM((C, N_FEATURES), jnp.float32),
            pltpu.SemaphoreType.DMA,
        ],
        compiler_params=pltpu.CompilerParams(use_tc_tiling_on_sc=False),
    )
    return run(x_flat, table, pos2d)


def kernel(x, text_embedding_weight, pos_embedding):
    bs, seq_len = x.shape
    x_flat = x.reshape(-1).astype(jnp.int32)
    pos2d = pos_embedding.reshape(-1, pos_embedding.shape[-1])[:seq_len]
    out = _embed(x_flat, text_embedding_weight, pos2d)
    return out.reshape(bs, seq_len, N_FEATURES)


# speedup vs baseline: 3.7808x; 1.2748x over previous
import jax
import jax.numpy as jnp
from jax import lax
from jax.experimental import pallas as pl
from jax.experimental.pallas import tpu as pltpu
from jax.experimental.pallas import tpu_sc as plsc

N_FEATURES = 64
BATCH = 4096
SEQ_LEN = 200

NC = 2
NS = 16
NW = NC * NS

N_FLAT = BATCH * SEQ_LEN
NPW = N_FLAT // NW
C = 128
G = NPW // C
POS_EXT = 3200
POS_PERIOD = POS_EXT // C
NB = 4
OUTER = (G + 3 + NB - 1) // NB


def _body(x_ref, tab_ref, pos_ref, out_ref, pos_ext, idx_v, buf,
          sem_pre, sem_idx, sem_gat, sem_out):
    c = lax.axis_index("c")
    s = lax.axis_index("s")
    wid = s * NC + c
    pltpu.sync_copy(pos_ref, pos_ext.at[pl.ds(s * SEQ_LEN, SEQ_LEN)])
    plsc.subcore_barrier()
    base0 = wid * NPW

    def pre_copies(h, b):
        off = lax.rem(h, POS_PERIOD) * C
        return (
            pltpu.make_async_copy(pos_ext.at[pl.ds(off, C)], buf.at[b], sem_pre),
            pltpu.make_async_copy(x_ref.at[pl.ds(base0 + h * C, C)],
                                  idx_v.at[b], sem_idx),
        )

    def gat_copy(b):
        return pltpu.make_async_copy(tab_ref.at[idx_v.at[b]], buf.at[b], sem_gat)

    def out_copy(h, b):
        return pltpu.make_async_copy(buf.at[b],
                                     out_ref.at[pl.ds(base0 + h * C, C)], sem_out)

    @pl.loop(0, OUTER)
    def _(o):
        for b in range(NB):
            t = o * NB + b

            h_w = t - 3
            @pl.when(jnp.logical_and(h_w >= 0, h_w < G))
            def _():
                out_copy(h_w, (b - 3) % NB).wait()

            h_s = t - 2
            @pl.when(jnp.logical_and(h_s >= 0, h_s < G))
            def _():
                bb = (b - 2) % NB
                gat_copy(bb).wait()
                out_copy(h_s, bb).start()

            h_g = t - 1
            @pl.when(jnp.logical_and(h_g >= 0, h_g < G))
            def _():
                bb = (b - 1) % NB
                cp, ci = pre_copies(h_g, bb)
                cp.wait()
                ci.wait()
                gat_copy(bb).start(add=True)

            @pl.when(t < G)
            def _():
                cp, ci = pre_copies(t, b)
                cp.start()
                ci.start()


@jax.jit
def _embed(x_flat, table, pos2d):
    run = pl.kernel(
        _body,
        out_type=jax.ShapeDtypeStruct((N_FLAT, N_FEATURES), jnp.float32),
        mesh=plsc.VectorSubcoreMesh(core_axis_name="c", subcore_axis_name="s"),
        scratch_types=[
            pltpu.VMEM_SHARED((POS_EXT, N_FEATURES), jnp.float32),
            pltpu.VMEM((NB, C), jnp.int32),
            pltpu.VMEM((NB, C, N_FEATURES), jnp.float32),
            pltpu.SemaphoreType.DMA,
            pltpu.SemaphoreType.DMA,
            pltpu.SemaphoreType.DMA,
            pltpu.SemaphoreType.DMA,
        ],
        compiler_params=pltpu.CompilerParams(use_tc_tiling_on_sc=False),
    )
    return run(x_flat, table, pos2d)


def kernel(x, text_embedding_weight, pos_embedding):
    bs, seq_len = x.shape
    x_flat = x.reshape(-1).astype(jnp.int32)
    pos2d = pos_embedding.reshape(-1, pos_embedding.shape[-1])[:seq_len]
    out = _embed(x_flat, text_embedding_weight, pos2d)
    return out.reshape(bs, seq_len, N_FEATURES)
